# baseline (speedup 1.0000x reference)
import jax
import jax.numpy as jnp
from jax import lax
from jax.experimental import pallas as pl
from jax.experimental.pallas import tpu as pltpu
from jax.experimental.pallas import tpu_sc as plsc

NUM_NODE = 10000
D = 128
B = 256
L = 20
E = 320000
LEAK = 0.2

NC = 2
NS = 16
NW = NC * NS
NPAD = 10240
DUMMY = NUM_NODE + 7
CH = 128
EPW = 10240
EPAD = NW * EPW
NCHUNK = EPW // CH
L_PAD = 32
BL = B * L
NBLK = BL // 128


def _tc_proj(feats, W, bias):
    def body(f_ref, w_ref, b_ref, o_ref):
        o_ref[...] = (
            jnp.dot(f_ref[...], w_ref[...], preferred_element_type=jnp.float32)
            + b_ref[...]
        )

    return pl.pallas_call(
        body,
        grid=(NPAD // 320,),
        in_specs=[
            pl.BlockSpec((320, D), lambda i: (i, 0)),
            pl.BlockSpec((D, D), lambda i: (0, 0)),
            pl.BlockSpec((1, D), lambda i: (0, 0)),
        ],
        out_specs=pl.BlockSpec((320, D), lambda i: (i, 0)),
        out_shape=jax.ShapeDtypeStruct((NPAD, D), jnp.float32),
    )(feats, W, bias)


def _sc_graph(x_hbm, pack_hbm, wg_hbm, ids2_hbm, item_hbm,
              agg_out, deg_out, nemb_out,
              agg_s, deg_s, packb, wgrp, rows2, ids_v,
              gdeg_v, semg0, semg1, sems0, sems1, semi0, semi1):
    c = lax.axis_index("c")
    s = lax.axis_index("s")
    wid = s * NC + c
    semg = (semg0, semg1)
    sems = (sems0, sems1)

    def zrow(r, _):
        for k in range(8):
            rows2[0, r, pl.ds(k * 16, 16)] = jnp.zeros((16,), jnp.float32)
        return 0

    lax.fori_loop(0, 128, zrow, 0)
    for k in range(8):
        gdeg_v[pl.ds(k * 16, 16)] = jnp.zeros((16,), jnp.float32)
    base_r = s * (NPAD // NS)
    for i in range(NPAD // NS // 128):
        pltpu.sync_copy(rows2.at[0], agg_s.at[pl.ds(base_r + i * 128, 128)])
        pltpu.sync_copy(gdeg_v, deg_s.at[pl.ds(base_r + i * 128, 128)])
    plsc.subcore_barrier()

    GRP = NCHUNK // 4
    semi = (semi0, semi1)

    def issue_gather(b, gb, q):
        for h in range(2):
            pltpu.async_copy(
                x_hbm.at[packb.at[gb, q, 0, pl.ds(h * 64, 64)]],
                rows2.at[b, pl.ds(h * 64, 64)], semg[b])

    pltpu.sync_copy(pack_hbm.at[wid * GRP], packb.at[0])
    pltpu.sync_copy(wg_hbm.at[wid * GRP], wgrp.at[0])
    pltpu.async_copy(pack_hbm.at[wid * GRP + 1], packb.at[1], semi1)
    pltpu.async_copy(wg_hbm.at[wid * GRP + 1], wgrp.at[1], semi1)
    issue_gather(0, 0, 0)
    issue_gather(1, 0, 1)

    def group(g2_, _):
      for gb in range(2):
        g = g2_ * 2 + gb
        for q in range(4):
            b = q % 2
            t = g * 4 + q
            for h in range(2):
                pltpu.make_async_copy(
                    x_hbm.at[packb.at[gb, q, 0, pl.ds(h * 64, 64)]],
                    rows2.at[b, pl.ds(h * 64, 64)], semg[b]).wait()

            def mrow(g2, _, b=b, gb=gb, q=q):
                wv = wgrp[gb, q, pl.ds(g2 * 16, 16)]
                for j in range(16):
                    r = g2 * 16 + j
                    wb = jnp.full((16,), wv[j], jnp.float32)
                    for k in range(8):
                        rows2[b, r, pl.ds(k * 16, 16)] = (
                            rows2[b, r, pl.ds(k * 16, 16)] * wb)
                return 0

            lax.fori_loop(0, CH // 16, mrow, 0)
            pltpu.async_copy(
                rows2.at[b], agg_s.at[packb.at[gb, q, 1]], sems[b],
                add=True)
            pltpu.async_copy(
                wgrp.at[gb, q], deg_s.at[packb.at[gb, q, 1]], sems[b],
                add=True)

            @pl.when(t + 2 < NCHUNK)
            def _(b=b, gb=gb, q=q):
                pltpu.make_async_copy(
                    rows2.at[b], agg_s.at[packb.at[gb, q, 1]],
                    sems[b]).wait()
                pltpu.make_async_copy(
                    wgrp.at[gb, q], deg_s.at[packb.at[gb, q, 1]],
                    sems[b]).wait()
                if q == 2:
                    pltpu.make_async_copy(
                        pack_hbm.at[0], packb.at[1 - gb],
                        semi[1 - gb]).wait()
                    pltpu.make_async_copy(
                        wg_hbm.at[0], wgrp.at[1 - gb],
                        semi[1 - gb]).wait()
                if q < 2:
                    issue_gather(b, gb, q + 2)
                else:
                    issue_gather(b, 1 - gb, q - 2)

            if q == 3:
                @pl.when(g + 2 < GRP)
                def _(g=g, gb=gb):
                    pltpu.async_copy(
                        pack_hbm.at[wid * GRP + g + 2],
                        packb.at[gb], semi[gb])
                    pltpu.async_copy(
                        wg_hbm.at[wid * GRP + g + 2],
                        wgrp.at[gb], semi[gb])
      return 0

    lax.fori_loop(0, GRP // 2, group, 0)
    for b in range(2):
        q = 2 + b
        pltpu.make_async_copy(
            rows2.at[b], agg_s.at[packb.at[1, q, 1]], sems[b]).wait()
        pltpu.make_async_copy(
            wgrp.at[1, q], deg_s.at[packb.at[1, q, 1]], sems[b]).wait()
    plsc.subcore_barrier()

    for j in range(3):
        blk = s + NS * j

        @pl.when(blk < NBLK)
        def _(blk=blk):
            pltpu.sync_copy(ids2_hbm.at[blk], ids_v.at[0])
            pltpu.async_copy(
                agg_s.at[ids_v.at[0]], rows2.at[0], semg0).wait()
            pltpu.sync_copy(
                rows2.at[0], agg_out.at[c, pl.ds(blk * 128, 128)])
            pltpu.async_copy(deg_s.at[ids_v.at[0]], gdeg_v, semg0).wait()
            pltpu.sync_copy(gdeg_v, deg_out.at[c, pl.ds(blk * 128, 128)])
            pltpu.async_copy(
                item_hbm.at[ids_v.at[0]], rows2.at[0], semg0).wait()
            pltpu.sync_copy(
                rows2.at[0], nemb_out.at[pl.ds(blk * 128, 128)])


def _sc_call(x, pack, wg, ids2, item_emb):
    mesh = plsc.VectorSubcoreMesh(
        core_axis_name="c", subcore_axis_name="s", num_cores=NC,
        num_subcores=NS)
    f = pl.kernel(
        _sc_graph,
        out_type=(
            jax.ShapeDtypeStruct((NC, BL, D), jnp.float32),
            jax.ShapeDtypeStruct((NC, BL), jnp.float32),
            jax.ShapeDtypeStruct((BL, D), jnp.float32),
        ),
        mesh=mesh,
        scratch_types=[
            pltpu.VMEM_SHARED((NPAD, D), jnp.float32),
            pltpu.VMEM_SHARED((NPAD,), jnp.float32),
            pltpu.VMEM((2, 4, 2, 128), jnp.int32),
            pltpu.VMEM((2, 4, 128), jnp.float32),
            pltpu.VMEM((2, CH, D), jnp.float32),
            pltpu.VMEM((1, 128), jnp.int32),
            pltpu.VMEM((128,), jnp.float32),
            pltpu.SemaphoreType.DMA,
            pltpu.SemaphoreType.DMA,
            pltpu.SemaphoreType.DMA,
            pltpu.SemaphoreType.DMA,
            pltpu.SemaphoreType.DMA,
            pltpu.SemaphoreType.DMA,
        ],
    )
    return f(x, pack, wg, ids2, item_emb)


def _tc_combine(h_pad, adj_pad, a_stack, agg2, deg2r, gW, gb):
    def body(h_ref, adj_ref, a_ref, agg_ref, deg_ref, w_ref, b_ref, o_ref):
        h = h_ref[0]
        adj = adj_ref[0]
        col = lax.broadcasted_iota(jnp.int32, (L_PAD, L_PAD), 1)
        alpha = jnp.where(col < L, jnp.float32(-9e15), jnp.float32(-jnp.inf))
        for k in range(4):
            ak = a_ref[k, :]
            e = lax.dot_general(
                h * ak[None, :], h, (((1,), (1,)), ((), ())),
                preferred_element_type=jnp.float32)
            e = jnp.where(e > 0, e, LEAK * e)
            alpha = jnp.where(adj == (k + 1), e, alpha)
        m = jnp.max(alpha, axis=1, keepdims=True)
        p = jnp.exp(alpha - m)
        alpha = p / jnp.sum(p, axis=1, keepdims=True)
        local = jnp.dot(alpha, h, preferred_element_type=jnp.float32)
        aggv = agg_ref[0, 0] + agg_ref[1, 0]
        degv = deg_ref[0, 0, 0] + deg_ref[1, 0, 0]
        outv = aggv / jnp.maximum(degv, 1e-9)[:, None]
        g = jnp.dot(outv, w_ref[...], preferred_element_type=jnp.float32)
        g = jnp.maximum(g + b_ref[...], 0.0)
        o_ref[0] = local[:L] + g

    return pl.pallas_call(
        body,
        grid=(B,),
        in_specs=[
            pl.BlockSpec((1, L_PAD, D), lambda b: (b, 0, 0)),
            pl.BlockSpec((1, L_PAD, L_PAD), lambda b: (b, 0, 0)),
            pl.BlockSpec((4, D), lambda b: (0, 0)),
            pl.BlockSpec((NC, 1, L, D), lambda b: (0, b, 0, 0)),
            pl.BlockSpec((NC, 1, 1, L), lambda b: (0, b, 0, 0)),
            pl.BlockSpec((D, D), lambda b: (0, 0)),
            pl.BlockSpec((1, D), lambda b: (0, 0)),
        ],
        out_specs=pl.BlockSpec((1, L, D), lambda b: (b, 0, 0)),
        out_shape=jax.ShapeDtypeStruct((B, L, D), jnp.float32),
    )(h_pad, adj_pad, a_stack, agg2, deg2r, gW, gb)


def kernel(inputs, local_adj, mask_item, unused_seq_ids, item_emb, a0, a1, a2,
           a3, features, edge_index, edge_weight, proj_W, proj_b, gnn_W,
           gnn_b):
    feats = jnp.zeros((NPAD, D), jnp.float32).at[:NUM_NODE + 1].set(features)
    x = _tc_proj(feats, proj_W, proj_b.reshape(1, D))

    src = jnp.concatenate(
        [edge_index[0].astype(jnp.int32), jnp.zeros((EPAD - E,), jnp.int32)])
    dst = jnp.concatenate(
        [edge_index[1].astype(jnp.int32),
         jnp.full((EPAD - E,), DUMMY, jnp.int32)])
    wgt = jnp.concatenate([edge_weight, jnp.zeros((EPAD - E,), jnp.float32)])
    pack = jnp.stack(
        [src.reshape(EPAD // 128, 128), dst.reshape(EPAD // 128, 128)],
        axis=1).reshape(EPAD // 512, 4, 2, 128)
    wg = wgt.reshape(EPAD // 512, 4, 128)
    ids2 = inputs.reshape(-1).astype(jnp.int32).reshape(NBLK, 128)

    agg2, deg2, nemb = _sc_call(x, pack, wg, ids2, item_emb)

    h_pad = jnp.zeros((B, L_PAD, D), jnp.float32).at[:, :L].set(
        nemb.reshape(B, L, D))
    adj_pad = jnp.zeros((B, L_PAD, L_PAD), jnp.int32).at[:, :L, :L].set(
        local_adj.astype(jnp.int32))
    a_stack = jnp.concatenate([a0, a1, a2, a3], axis=1).T
    agg2 = agg2.reshape(NC, B, L, D)
    deg2r = deg2.reshape(NC, B, 1, L)

    return _tc_combine(h_pad, adj_pad, a_stack, agg2, deg2r, gnn_W,
                       gnn_b.reshape(1, D))


# speedup vs baseline: 1.0008x; 1.0008x over previous
import jax
import jax.numpy as jnp
from jax import lax
from jax.experimental import pallas as pl
from jax.experimental.pallas import tpu as pltpu
from jax.experimental.pallas import tpu_sc as plsc

NUM_NODE = 10000
D = 128
B = 256
L = 20
E = 320000
LEAK = 0.2

NC = 2
NS = 16
NW = NC * NS
NPAD = 10240
DUMMY = NUM_NODE + 7
CH = 128
EPW = 10240
EPAD = NW * EPW
NCHUNK = EPW // CH
L_PAD = 32
BL = B * L
NBLK = BL // 128


def _tc_proj(feats, W, bias):
    def body(f_ref, w_ref, b_ref, o_ref):
        o_ref[...] = (
            jnp.dot(f_ref[...], w_ref[...], preferred_element_type=jnp.float32)
            + b_ref[...]
        )

    return pl.pallas_call(
        body,
        grid=(NPAD // 320,),
        in_specs=[
            pl.BlockSpec((320, D), lambda i: (i, 0)),
            pl.BlockSpec((D, D), lambda i: (0, 0)),
            pl.BlockSpec((1, D), lambda i: (0, 0)),
        ],
        out_specs=pl.BlockSpec((320, D), lambda i: (i, 0)),
        out_shape=jax.ShapeDtypeStruct((NPAD, D), jnp.float32),
    )(feats, W, bias)


def _sc_graph(x_hbm, pack_hbm, wg_hbm, ids2_hbm, item_hbm,
              agg_out, deg_out, nemb_out,
              agg_s, deg_s, packb, wgrp, rows2, ids_v,
              gdeg_v, semg0, semg1, sems0, sems1, semi0, semi1):
    c = lax.axis_index("c")
    s = lax.axis_index("s")
    wid = s * NC + c
    semg = (semg0, semg1)
    sems = (sems0, sems1)

    def zrow(r, _):
        for k in range(8):
            rows2[0, r, pl.ds(k * 16, 16)] = jnp.zeros((16,), jnp.float32)
        return 0

    lax.fori_loop(0, 128, zrow, 0)
    for k in range(8):
        gdeg_v[pl.ds(k * 16, 16)] = jnp.zeros((16,), jnp.float32)
    base_r = s * (NPAD // NS)
    for i in range(NPAD // NS // 128):
        pltpu.sync_copy(rows2.at[0], agg_s.at[pl.ds(base_r + i * 128, 128)])
        pltpu.sync_copy(gdeg_v, deg_s.at[pl.ds(base_r + i * 128, 128)])
    plsc.subcore_barrier()

    GRP = NCHUNK // 4
    semi = (semi0, semi1)

    def issue_gather(b, gb, q):
        pltpu.async_copy(
            x_hbm.at[packb.at[gb, q, 0]], rows2.at[b], semg[b])

    pltpu.sync_copy(pack_hbm.at[wid * GRP], packb.at[0])
    pltpu.sync_copy(wg_hbm.at[wid * GRP], wgrp.at[0])
    pltpu.async_copy(pack_hbm.at[wid * GRP + 1], packb.at[1], semi1)
    pltpu.async_copy(wg_hbm.at[wid * GRP + 1], wgrp.at[1], semi1)
    issue_gather(0, 0, 0)
    issue_gather(1, 0, 1)

    def group(g2_, _):
      for gb in range(2):
        g = g2_ * 2 + gb
        for q in range(4):
            b = q % 2
            t = g * 4 + q
            pltpu.make_async_copy(
                x_hbm.at[packb.at[gb, q, 0]], rows2.at[b], semg[b]).wait()

            def mrow(g2, _, b=b, gb=gb, q=q):
                wv = wgrp[gb, q, pl.ds(g2 * 16, 16)]
                for j in range(16):
                    r = g2 * 16 + j
                    wb = jnp.full((16,), wv[j], jnp.float32)
                    for k in range(8):
                        rows2[b, r, pl.ds(k * 16, 16)] = (
                            rows2[b, r, pl.ds(k * 16, 16)] * wb)
                return 0

            lax.fori_loop(0, CH // 16, mrow, 0)
            pltpu.async_copy(
                rows2.at[b], agg_s.at[packb.at[gb, q, 1]], sems[b],
                add=True)
            pltpu.async_copy(
                wgrp.at[gb, q], deg_s.at[packb.at[gb, q, 1]], sems[b],
                add=True)

            @pl.when(t + 2 < NCHUNK)
            def _(b=b, gb=gb, q=q):
                pltpu.make_async_copy(
                    rows2.at[b], agg_s.at[packb.at[gb, q, 1]],
                    sems[b]).wait()
                pltpu.make_async_copy(
                    wgrp.at[gb, q], deg_s.at[packb.at[gb, q, 1]],
                    sems[b]).wait()
                if q == 2:
                    pltpu.make_async_copy(
                        pack_hbm.at[0], packb.at[1 - gb],
                        semi[1 - gb]).wait()
                    pltpu.make_async_copy(
                        wg_hbm.at[0], wgrp.at[1 - gb],
                        semi[1 - gb]).wait()
                if q < 2:
                    issue_gather(b, gb, q + 2)
                else:
                    issue_gather(b, 1 - gb, q - 2)

            if q == 3:
                @pl.when(g + 2 < GRP)
                def _(g=g, gb=gb):
                    pltpu.async_copy(
                        pack_hbm.at[wid * GRP + g + 2],
                        packb.at[gb], semi[gb])
                    pltpu.async_copy(
                        wg_hbm.at[wid * GRP + g + 2],
                        wgrp.at[gb], semi[gb])
      return 0

    lax.fori_loop(0, GRP // 2, group, 0)
    for b in range(2):
        q = 2 + b
        pltpu.make_async_copy(
            rows2.at[b], agg_s.at[packb.at[1, q, 1]], sems[b]).wait()
        pltpu.make_async_copy(
            wgrp.at[1, q], deg_s.at[packb.at[1, q, 1]], sems[b]).wait()
    plsc.subcore_barrier()

    for j in range(3):
        blk = s + NS * j

        @pl.when(blk < NBLK)
        def _(blk=blk):
            pltpu.sync_copy(ids2_hbm.at[blk], ids_v.at[0])
            pltpu.async_copy(
                agg_s.at[ids_v.at[0]], rows2.at[0], semg0).wait()
            pltpu.sync_copy(
                rows2.at[0], agg_out.at[c, pl.ds(blk * 128, 128)])
            pltpu.async_copy(deg_s.at[ids_v.at[0]], gdeg_v, semg0).wait()
            pltpu.sync_copy(gdeg_v, deg_out.at[c, pl.ds(blk * 128, 128)])
            pltpu.async_copy(
                item_hbm.at[ids_v.at[0]], rows2.at[0], semg0).wait()
            pltpu.sync_copy(
                rows2.at[0], nemb_out.at[pl.ds(blk * 128, 128)])


def _sc_call(x, pack, wg, ids2, item_emb):
    mesh = plsc.VectorSubcoreMesh(
        core_axis_name="c", subcore_axis_name="s", num_cores=NC,
        num_subcores=NS)
    f = pl.kernel(
        _sc_graph,
        out_type=(
            jax.ShapeDtypeStruct((NC, BL, D), jnp.float32),
            jax.ShapeDtypeStruct((NC, BL), jnp.float32),
            jax.ShapeDtypeStruct((BL, D), jnp.float32),
        ),
        mesh=mesh,
        scratch_types=[
            pltpu.VMEM_SHARED((NPAD, D), jnp.float32),
            pltpu.VMEM_SHARED((NPAD,), jnp.float32),
            pltpu.VMEM((2, 4, 2, 128), jnp.int32),
            pltpu.VMEM((2, 4, 128), jnp.float32),
            pltpu.VMEM((2, CH, D), jnp.float32),
            pltpu.VMEM((1, 128), jnp.int32),
            pltpu.VMEM((128,), jnp.float32),
            pltpu.SemaphoreType.DMA,
            pltpu.SemaphoreType.DMA,
            pltpu.SemaphoreType.DMA,
            pltpu.SemaphoreType.DMA,
            pltpu.SemaphoreType.DMA,
            pltpu.SemaphoreType.DMA,
        ],
    )
    return f(x, pack, wg, ids2, item_emb)


def _tc_combine(h_pad, adj_pad, a_stack, agg2, deg2r, gW, gb):
    def body(h_ref, adj_ref, a_ref, agg_ref, deg_ref, w_ref, b_ref, o_ref):
        h = h_ref[0]
        adj = adj_ref[0]
        col = lax.broadcasted_iota(jnp.int32, (L_PAD, L_PAD), 1)
        alpha = jnp.where(col < L, jnp.float32(-9e15), jnp.float32(-jnp.inf))
        for k in range(4):
            ak = a_ref[k, :]
            e = lax.dot_general(
                h * ak[None, :], h, (((1,), (1,)), ((), ())),
                preferred_element_type=jnp.float32)
            e = jnp.where(e > 0, e, LEAK * e)
            alpha = jnp.where(adj == (k + 1), e, alpha)
        m = jnp.max(alpha, axis=1, keepdims=True)
        p = jnp.exp(alpha - m)
        alpha = p / jnp.sum(p, axis=1, keepdims=True)
        local = jnp.dot(alpha, h, preferred_element_type=jnp.float32)
        aggv = agg_ref[0, 0] + agg_ref[1, 0]
        degv = deg_ref[0, 0, 0] + deg_ref[1, 0, 0]
        outv = aggv / jnp.maximum(degv, 1e-9)[:, None]
        g = jnp.dot(outv, w_ref[...], preferred_element_type=jnp.float32)
        g = jnp.maximum(g + b_ref[...], 0.0)
        o_ref[0] = local[:L] + g

    return pl.pallas_call(
        body,
        grid=(B,),
        in_specs=[
            pl.BlockSpec((1, L_PAD, D), lambda b: (b, 0, 0)),
            pl.BlockSpec((1, L_PAD, L_PAD), lambda b: (b, 0, 0)),
            pl.BlockSpec((4, D), lambda b: (0, 0)),
            pl.BlockSpec((NC, 1, L, D), lambda b: (0, b, 0, 0)),
            pl.BlockSpec((NC, 1, 1, L), lambda b: (0, b, 0, 0)),
            pl.BlockSpec((D, D), lambda b: (0, 0)),
            pl.BlockSpec((1, D), lambda b: (0, 0)),
        ],
        out_specs=pl.BlockSpec((1, L, D), lambda b: (b, 0, 0)),
        out_shape=jax.ShapeDtypeStruct((B, L, D), jnp.float32),
    )(h_pad, adj_pad, a_stack, agg2, deg2r, gW, gb)


def kernel(inputs, local_adj, mask_item, unused_seq_ids, item_emb, a0, a1, a2,
           a3, features, edge_index, edge_weight, proj_W, proj_b, gnn_W,
           gnn_b):
    feats = jnp.zeros((NPAD, D), jnp.float32).at[:NUM_NODE + 1].set(features)
    x = _tc_proj(feats, proj_W, proj_b.reshape(1, D))

    src = jnp.concatenate(
        [edge_index[0].astype(jnp.int32), jnp.zeros((EPAD - E,), jnp.int32)])
    dst = jnp.concatenate(
        [edge_index[1].astype(jnp.int32),
         jnp.full((EPAD - E,), DUMMY, jnp.int32)])
    wgt = jnp.concatenate([edge_weight, jnp.zeros((EPAD - E,), jnp.float32)])
    pack = jnp.stack(
        [src.reshape(EPAD // 128, 128), dst.reshape(EPAD // 128, 128)],
        axis=1).reshape(EPAD // 512, 4, 2, 128)
    wg = wgt.reshape(EPAD // 512, 4, 128)
    ids2 = inputs.reshape(-1).astype(jnp.int32).reshape(NBLK, 128)

    agg2, deg2, nemb = _sc_call(x, pack, wg, ids2, item_emb)

    h_pad = jnp.zeros((B, L_PAD, D), jnp.float32).at[:, :L].set(
        nemb.reshape(B, L, D))
    adj_pad = jnp.zeros((B, L_PAD, L_PAD), jnp.int32).at[:, :L, :L].set(
        local_adj.astype(jnp.int32))
    a_stack = jnp.concatenate([a0, a1, a2, a3], axis=1).T
    agg2 = agg2.reshape(NC, B, L, D)
    deg2r = deg2.reshape(NC, B, 1, L)

    return _tc_combine(h_pad, adj_pad, a_stack, agg2, deg2r, gnn_W,
                       gnn_b.reshape(1, D))
